# baseline (speedup 1.0000x reference)
import functools

import jax
import jax.numpy as jnp
from jax import lax
from jax.experimental import pallas as pl
from jax.experimental.pallas import tpu as pltpu
from jax.experimental.pallas import tpu_sc as plsc

BATCH = 4096
HIST = 200
DIM = 32
VOCAB = 1000000
TDIM = 128
RPS = TDIM // DIM
TROWS = VOCAB * DIM // TDIM
N = BATCH * HIST
NC = 2
NS = 16
NW = NC * NS
PER_W = N // NW
CHUNK = 128
NCHUNK = PER_W // CHUNK
GROUPS = CHUNK // 16
OUT_W = 2 * DIM


def _sc_body(idx_hbm, wr_hbm, wi_hbm, out_hbm, idx_v, off_v, r_v, i_v, o_v,
             sem_r, sem_i):
    wid = lax.axis_index("s") * NC + lax.axis_index("c")
    iota = lax.iota(jnp.int32, 16)

    def body(g, carry):
        base = wid * PER_W + g * CHUNK
        pltpu.sync_copy(idx_hbm.at[pl.ds(base, CHUNK)], idx_v)
        for v in range(GROUPS):
            sl = pl.ds(v * 16, 16)
            ids = idx_v[sl]
            idx_v[sl] = lax.shift_right_logical(ids, 2)
            off_v[sl] = (ids & (RPS - 1)) * DIM
        cr = pltpu.async_copy(wr_hbm.at[idx_v], r_v, sem_r)
        ci = pltpu.async_copy(wi_hbm.at[idx_v], i_v, sem_i)
        cr.wait()
        ci.wait()

        def groups(grp, carry2):
            rows = grp * 16 + iota
            offv = off_v[pl.ds(grp * 16, 16)]
            obase = grp * (16 * OUT_W) + iota * OUT_W
            for d in range(DIM):
                colv = offv + d
                ocol = obase + 2 * d
                rvals = plsc.load_gather(r_v, [rows, colv])
                plsc.store_scatter(o_v, [ocol], rvals)
                ivals = plsc.load_gather(i_v, [rows, colv])
                plsc.store_scatter(o_v, [ocol + 1], ivals)
            return carry2

        lax.fori_loop(0, GROUPS, groups, 0)
        pltpu.sync_copy(o_v, out_hbm.at[pl.ds(base * OUT_W, CHUNK * OUT_W)])
        return carry

    lax.fori_loop(0, NCHUNK, body, 0)


@jax.jit
def _complex_embed(ids, W_real, W_imag):
    run = pl.kernel(
        _sc_body,
        out_type=jax.ShapeDtypeStruct((N * OUT_W,), jnp.float32),
        mesh=plsc.VectorSubcoreMesh(core_axis_name="c", subcore_axis_name="s"),
        compiler_params=pltpu.CompilerParams(
            use_tc_tiling_on_sc=True, needs_layout_passes=False),
        scratch_types=[
            pltpu.VMEM((CHUNK,), jnp.int32),
            pltpu.VMEM((CHUNK,), jnp.int32),
            pltpu.VMEM((CHUNK, TDIM), jnp.float32),
            pltpu.VMEM((CHUNK, TDIM), jnp.float32),
            pltpu.VMEM((CHUNK * OUT_W,), jnp.float32),
            pltpu.SemaphoreType.DMA,
            pltpu.SemaphoreType.DMA,
        ],
    )
    return run(ids, W_real, W_imag)


def kernel(token_ids, W_real, W_imag):
    ids = token_ids.reshape(N).astype(jnp.int32)
    out = _complex_embed(ids, W_real.reshape(TROWS, TDIM),
                         W_imag.reshape(TROWS, TDIM))
    return out.reshape(BATCH, HIST, DIM, 2)


# speedup vs baseline: 5.8195x; 5.8195x over previous
import functools

import jax
import jax.numpy as jnp
from jax import lax
from jax.experimental import pallas as pl
from jax.experimental.pallas import tpu as pltpu
from jax.experimental.pallas import tpu_sc as plsc

BATCH = 4096
HIST = 200
DIM = 32
VOCAB = 1000000
N = BATCH * HIST
NC = 2
NS = 16
NW = NC * NS
QROWS = VOCAB // 2
QDIM = 4 * DIM
NBLK = VOCAB // 128
BLK_ITERS = NBLK // NW + 1
HB = HIST // 8
KB = BATCH // 128
OUT_ROWS = HIST * DIM * KB
OUT_W = 256


def _fmt_body(wr_hbm, wi_hbm, wrt_hbm, wit_hbm, wri_hbm, av, bv, ov, sem):
    wid = lax.axis_index("s") * NC + lax.axis_index("c")
    iota = lax.iota(jnp.int32, 16)
    rowv = []
    colv = []
    for v in range(8):
        l = iota + 16 * v
        rowv.append(lax.shift_right_logical(l, 1))
        colv.append((l & 1) * 64)

    def permute_block():
        def dloop(d, carry):
            for v in range(8):
                rv = av[d, pl.ds(16 * v, 16)]
                plsc.store_scatter(ov, [rowv[v], colv[v] + d], rv)
                iv = bv[d, pl.ds(16 * v, 16)]
                plsc.store_scatter(ov, [rowv[v], colv[v] + (d + DIM)], iv)
            return carry
        lax.fori_loop(0, DIM, dloop, 0)

    def body(i, carry):
        j = wid + NW * i

        @pl.when(j < NBLK)
        def _():
            pltpu.sync_copy(wr_hbm.at[:, pl.ds(j * 128, 128)], av)
            pltpu.sync_copy(wi_hbm.at[:, pl.ds(j * 128, 128)], bv)
            permute_block()
            pltpu.sync_copy(ov, wri_hbm.at[pl.ds(j * 64, 64)])
        return carry

    lax.fori_loop(0, BLK_ITERS, body, 0)

    @pl.when(wid == 0)
    def _tail():
        pltpu.sync_copy(wrt_hbm.at[:], av)
        pltpu.sync_copy(wit_hbm.at[:], bv)
        permute_block()
        pltpu.sync_copy(ov, wri_hbm.at[pl.ds((VOCAB - 128) // 2, 64)])


def _gather_body(ids_hbm, wri_hbm, out_hbm, idt, idxg, offv, gv, ov,
                 sem_g, sem_o):
    wid = lax.axis_index("s") * NC + lax.axis_index("c")
    iota = lax.iota(jnp.int32, 16)

    def item(i, carry):
        pltpu.sync_copy(
            ids_hbm.at[pl.ds(8 * i, 8), pl.ds(wid * 128, 128)], idt)

        def per_h(hh, c2):
            for g in range(8):
                t = idt[hh, pl.ds(16 * g, 16)]
                idxg[pl.ds(16 * g, 16)] = lax.shift_right_logical(t, 1)
                offv[pl.ds(16 * g, 16)] = (t & 1) * (2 * DIM)
            pltpu.async_copy(wri_hbm.at[idxg], gv, sem_g).wait()

            def grp(g, c3):
                rows = g * 16 + iota
                off = offv[pl.ds(16 * g, 16)]
                for d in range(DIM):
                    for c in range(2):
                        vals = plsc.load_gather(gv, [rows, off + (c * DIM + d)])
                        ov[d, pl.ds(c * 128 + 16 * g, 16)] = vals
                return c3

            lax.fori_loop(0, 8, grp, 0)
            base = (8 * i + hh) * (DIM * KB) + wid
            c1 = pltpu.async_copy(ov.at[pl.ds(0, 16)],
                                  out_hbm.at[base + KB * iota], sem_o)
            c2_ = pltpu.async_copy(ov.at[pl.ds(16, 16)],
                                   out_hbm.at[base + 16 * KB + KB * iota],
                                   sem_o)
            c1.wait()
            c2_.wait()
            return c2

        lax.fori_loop(0, 8, per_h, 0)
        return carry

    lax.fori_loop(0, HB, item, 0)


@jax.jit
def _complex_embed(ids2, wr_t, wi_t, wr_tail, wi_tail):
    mesh = plsc.VectorSubcoreMesh(core_axis_name="c", subcore_axis_name="s")
    params = pltpu.CompilerParams(
        use_tc_tiling_on_sc=True, needs_layout_passes=False)

    wri = pl.kernel(
        _fmt_body,
        out_type=jax.ShapeDtypeStruct((QROWS, QDIM), jnp.float32),
        mesh=mesh,
        compiler_params=params,
        scratch_types=[
            pltpu.VMEM((DIM, 128), jnp.float32),
            pltpu.VMEM((DIM, 128), jnp.float32),
            pltpu.VMEM((64, QDIM), jnp.float32),
            pltpu.SemaphoreType.DMA,
        ],
    )(wr_t, wi_t, wr_tail, wi_tail)

    out2d = pl.kernel(
        _gather_body,
        out_type=jax.ShapeDtypeStruct((OUT_ROWS, OUT_W), jnp.float32),
        mesh=mesh,
        compiler_params=params,
        scratch_types=[
            pltpu.VMEM((8, 128), jnp.int32),
            pltpu.VMEM((128,), jnp.int32),
            pltpu.VMEM((128,), jnp.int32),
            pltpu.VMEM((128, QDIM), jnp.float32),
            pltpu.VMEM((DIM, OUT_W), jnp.float32),
            pltpu.SemaphoreType.DMA,
            pltpu.SemaphoreType.DMA,
        ],
    )(ids2, wri)
    return out2d


def kernel(token_ids, W_real, W_imag):
    wr_t = W_real.T
    wi_t = W_imag.T
    wr_tail = lax.slice(wr_t, (0, VOCAB - 128), (DIM, VOCAB))
    wi_tail = lax.slice(wi_t, (0, VOCAB - 128), (DIM, VOCAB))
    ids2 = token_ids.T.astype(jnp.int32)
    out2d = _complex_embed(ids2, wr_t, wi_t, wr_tail, wi_tail)
    out5 = out2d.reshape(HIST, DIM, KB, 2, 128)
    return out5.transpose(2, 4, 0, 1, 3).reshape(BATCH, HIST, DIM, 2)


# speedup vs baseline: 7.5753x; 1.3017x over previous
import functools

import jax
import jax.numpy as jnp
from jax import lax
from jax.experimental import pallas as pl
from jax.experimental.pallas import tpu as pltpu
from jax.experimental.pallas import tpu_sc as plsc

BATCH = 4096
HIST = 200
DIM = 32
VOCAB = 1000000
N = BATCH * HIST
NC = 2
NS = 16
NW = NC * NS
QROWS = VOCAB // 2
QDIM = 4 * DIM
WIDE = 256
NBLK = VOCAB // WIDE
FMT_IT = 124
HB = HIST // 8
KB = BATCH // 128
NU = HIST
OUT_ROWS = HIST * DIM * KB
OUT_W = 256


def _fmt_body(wr_hbm, wi_hbm, wrt_hbm, wit_hbm, wri_hbm,
              av0, av1, bv0, bv1, ov0, ov1, si0, si1, so0, so1):
    wid = lax.axis_index("s") * NC + lax.axis_index("c")
    iota = lax.iota(jnp.int32, 16)
    avs, bvs, ovs = (av0, av1), (bv0, bv1), (ov0, ov1)
    sis, sos = (si0, si1), (so0, so1)
    rowv, colv = [], []
    for v in range(WIDE // 16):
        l = iota + 16 * v
        rowv.append(lax.shift_right_logical(l, 1))
        colv.append((l & 1) * 64)

    def issue_in(i, p):
        j = wid + NW * i

        @pl.when(j < NBLK)
        def _():
            pltpu.async_copy(wr_hbm.at[:, pl.ds(j * WIDE, WIDE)], avs[p],
                             sis[p])
            pltpu.async_copy(wi_hbm.at[:, pl.ds(j * WIDE, WIDE)], bvs[p],
                             sis[p])

    def permute(a, b, o, nv):
        def dloop(d, carry):
            for v in range(nv):
                plsc.store_scatter(o, [rowv[v], colv[v] + d],
                                   a[d, pl.ds(16 * v, 16)])
                plsc.store_scatter(o, [rowv[v], colv[v] + (d + DIM)],
                                   b[d, pl.ds(16 * v, 16)])
            return carry
        lax.fori_loop(0, DIM, dloop, 0)

    issue_in(0, 0)

    def body(ii, carry):
        for p in (0, 1):
            i = 2 * ii + p
            issue_in(i + 1, (p + 1) % 2)
            j = wid + NW * i

            @pl.when(j < NBLK)
            def _():
                pltpu.make_async_copy(
                    wr_hbm.at[:, pl.ds(0, WIDE)], avs[p], sis[p]).wait()
                pltpu.make_async_copy(
                    wi_hbm.at[:, pl.ds(0, WIDE)], bvs[p], sis[p]).wait()

                @pl.when(i >= 2)
                def __():
                    pltpu.make_async_copy(
                        ovs[p], wri_hbm.at[pl.ds(0, WIDE // 2)],
                        sos[p]).wait()

                permute(avs[p], bvs[p], ovs[p], WIDE // 16)
                pltpu.async_copy(
                    ovs[p], wri_hbm.at[pl.ds(j * (WIDE // 2), WIDE // 2)],
                    sos[p])
        return carry

    lax.fori_loop(0, FMT_IT // 2, body, 0)
    pltpu.make_async_copy(ov0, wri_hbm.at[pl.ds(0, WIDE // 2)], so0).wait()
    pltpu.make_async_copy(ov1, wri_hbm.at[pl.ds(0, WIDE // 2)], so1).wait()

    @pl.when(wid == 0)
    def _tail():
        pltpu.sync_copy(wrt_hbm.at[:], av0.at[:, pl.ds(0, 128)])
        pltpu.sync_copy(wit_hbm.at[:], bv0.at[:, pl.ds(0, 128)])
        permute(av0, bv0, ov0, 8)
        pltpu.sync_copy(ov0.at[pl.ds(0, 64)],
                        wri_hbm.at[pl.ds((VOCAB - 128) // 2, 64)])


def _gather_body(ids_hbm, wri_hbm, out_hbm, idt, idx0, idx1, off0, off1,
                 gv0, gv1, ov0, ov1, sg0, sg1, so0, so1):
    wid = lax.axis_index("s") * NC + lax.axis_index("c")
    iota = lax.iota(jnp.int32, 16)
    idxs, offs = (idx0, idx1), (off0, off1)
    gvs, ovs = (gv0, gv1), (ov0, ov1)
    sgs, sos = (sg0, sg1), (so0, so1)

    def load_ids(i):
        pltpu.sync_copy(
            ids_hbm.at[pl.ds(8 * i, 8), pl.ds(wid * 128, 128)], idt)

    def prep_and_fire(u, p):
        @pl.when(u < NU)
        def _():
            hh = u & 7
            for g in range(8):
                t = idt[hh, pl.ds(16 * g, 16)]
                idxs[p][pl.ds(16 * g, 16)] = lax.shift_right_logical(t, 1)
                offs[p][pl.ds(16 * g, 16)] = (t & 1) * (2 * DIM)
            pltpu.async_copy(wri_hbm.at[idxs[p]], gvs[p], sgs[p])

    def interleave(p):
        def grp(g, c3):
            rows = g * 16 + iota
            off = offs[p][pl.ds(16 * g, 16)]
            for d in range(DIM):
                for c in range(2):
                    vals = plsc.load_gather(gvs[p], [rows, off + (c * DIM + d)])
                    ovs[p][d, pl.ds(c * 128 + 16 * g, 16)] = vals
            return c3
        lax.fori_loop(0, 8, grp, 0)

    def drain_out(p):
        pltpu.make_async_copy(ovs[p].at[pl.ds(0, 16)],
                              out_hbm.at[KB * iota], sos[p]).wait()
        pltpu.make_async_copy(ovs[p].at[pl.ds(16, 16)],
                              out_hbm.at[KB * iota], sos[p]).wait()

    load_ids(0)
    prep_and_fire(0, 0)

    def body(uu, carry):
        for p in (0, 1):
            u = 2 * uu + p
            nxt = u + 1

            @pl.when((nxt & 7) == 0)
            def _():
                @pl.when(nxt < NU)
                def __():
                    load_ids(lax.shift_right_logical(nxt, 3))

            prep_and_fire(nxt, (p + 1) % 2)
            pltpu.make_async_copy(wri_hbm.at[idxs[p]], gvs[p], sgs[p]).wait()

            @pl.when(u >= 2)
            def _():
                drain_out(p)

            interleave(p)
            base = u * (DIM * KB) + wid
            pltpu.async_copy(ovs[p].at[pl.ds(0, 16)],
                             out_hbm.at[base + KB * iota], sos[p])
            pltpu.async_copy(ovs[p].at[pl.ds(16, 16)],
                             out_hbm.at[base + 16 * KB + KB * iota], sos[p])
        return carry

    lax.fori_loop(0, NU // 2, body, 0)
    drain_out(0)
    drain_out(1)


@jax.jit
def _complex_embed(ids2, wr_t, wi_t, wr_tail, wi_tail):
    mesh = plsc.VectorSubcoreMesh(core_axis_name="c", subcore_axis_name="s")
    params = pltpu.CompilerParams(
        use_tc_tiling_on_sc=True, needs_layout_passes=False)

    wri = pl.kernel(
        _fmt_body,
        out_type=jax.ShapeDtypeStruct((QROWS, QDIM), jnp.float32),
        mesh=mesh,
        compiler_params=params,
        scratch_types=[
            pltpu.VMEM((DIM, WIDE), jnp.float32),
            pltpu.VMEM((DIM, WIDE), jnp.float32),
            pltpu.VMEM((DIM, WIDE), jnp.float32),
            pltpu.VMEM((DIM, WIDE), jnp.float32),
            pltpu.VMEM((WIDE // 2, QDIM), jnp.float32),
            pltpu.VMEM((WIDE // 2, QDIM), jnp.float32),
            pltpu.SemaphoreType.DMA,
            pltpu.SemaphoreType.DMA,
            pltpu.SemaphoreType.DMA,
            pltpu.SemaphoreType.DMA,
        ],
    )(wr_t, wi_t, wr_tail, wi_tail)

    out2d = pl.kernel(
        _gather_body,
        out_type=jax.ShapeDtypeStruct((OUT_ROWS, OUT_W), jnp.float32),
        mesh=mesh,
        compiler_params=params,
        scratch_types=[
            pltpu.VMEM((8, 128), jnp.int32),
            pltpu.VMEM((128,), jnp.int32),
            pltpu.VMEM((128,), jnp.int32),
            pltpu.VMEM((128,), jnp.int32),
            pltpu.VMEM((128,), jnp.int32),
            pltpu.VMEM((128, QDIM), jnp.float32),
            pltpu.VMEM((128, QDIM), jnp.float32),
            pltpu.VMEM((DIM, OUT_W), jnp.float32),
            pltpu.VMEM((DIM, OUT_W), jnp.float32),
            pltpu.SemaphoreType.DMA,
            pltpu.SemaphoreType.DMA,
            pltpu.SemaphoreType.DMA,
            pltpu.SemaphoreType.DMA,
        ],
    )(ids2, wri)
    return out2d


def kernel(token_ids, W_real, W_imag):
    wr_t = W_real.T
    wi_t = W_imag.T
    wr_tail = lax.slice(wr_t, (0, VOCAB - 128), (DIM, VOCAB))
    wi_tail = lax.slice(wi_t, (0, VOCAB - 128), (DIM, VOCAB))
    ids2 = token_ids.T.astype(jnp.int32)
    out2d = _complex_embed(ids2, wr_t, wi_t, wr_tail, wi_tail)
    out5 = out2d.reshape(HIST, DIM, KB, 2, 128)
    return out5.transpose(2, 4, 0, 1, 3).reshape(BATCH, HIST, DIM, 2)


# speedup vs baseline: 8.8549x; 1.1689x over previous
import functools

import jax
import jax.numpy as jnp
from jax import lax
from jax.experimental import pallas as pl
from jax.experimental.pallas import tpu as pltpu
from jax.experimental.pallas import tpu_sc as plsc

BATCH = 4096
HIST = 200
DIM = 32
VOCAB = 1000000
N = BATCH * HIST
NC = 2
NS = 16
NW = NC * NS
QROWS = VOCAB // 2
QDIM = 4 * DIM
WIDE = 256
NBLK = VOCAB // WIDE
FMT_IT = 124
HB = HIST // 8
KB = BATCH // 128
NU = HIST
OUT_ROWS = HIST * DIM * KB
OUT_W = 256


def _fmt_body(wr_hbm, wi_hbm, wrt_hbm, wit_hbm, wri_hbm,
              av0, av1, bv0, bv1, ov0, ov1, si0, si1, so0, so1):
    wid = lax.axis_index("s") * NC + lax.axis_index("c")
    iota = lax.iota(jnp.int32, 16)
    avs, bvs, ovs = (av0, av1), (bv0, bv1), (ov0, ov1)
    sis, sos = (si0, si1), (so0, so1)
    rowv, colv = [], []
    for v in range(WIDE // 16):
        l = iota + 16 * v
        rowv.append(lax.shift_right_logical(l, 1))
        colv.append((l & 1) * 64)

    def issue_in(i, p):
        j = wid + NW * i

        @pl.when(j < NBLK)
        def _():
            pltpu.async_copy(wr_hbm.at[:, pl.ds(j * WIDE, WIDE)], avs[p],
                             sis[p])
            pltpu.async_copy(wi_hbm.at[:, pl.ds(j * WIDE, WIDE)], bvs[p],
                             sis[p])

    def permute(a, b, o, nv):
        def dloop(d, carry):
            ra = [a[d, pl.ds(16 * v, 16)] for v in range(nv)]
            rb = [b[d, pl.ds(16 * v, 16)] for v in range(nv)]
            for v in range(nv):
                plsc.store_scatter(o, [rowv[v], colv[v] + d], ra[v])
            for v in range(nv):
                plsc.store_scatter(o, [rowv[v], colv[v] + (d + DIM)], rb[v])
            return carry
        lax.fori_loop(0, DIM, dloop, 0)

    issue_in(0, 0)

    def body(ii, carry):
        for p in (0, 1):
            i = 2 * ii + p
            issue_in(i + 1, (p + 1) % 2)
            j = wid + NW * i

            @pl.when(j < NBLK)
            def _():
                pltpu.make_async_copy(
                    wr_hbm.at[:, pl.ds(0, WIDE)], avs[p], sis[p]).wait()
                pltpu.make_async_copy(
                    wi_hbm.at[:, pl.ds(0, WIDE)], bvs[p], sis[p]).wait()

                @pl.when(i >= 2)
                def __():
                    pltpu.make_async_copy(
                        ovs[p], wri_hbm.at[pl.ds(0, WIDE // 2)],
                        sos[p]).wait()

                permute(avs[p], bvs[p], ovs[p], WIDE // 16)
                pltpu.async_copy(
                    ovs[p], wri_hbm.at[pl.ds(j * (WIDE // 2), WIDE // 2)],
                    sos[p])
        return carry

    lax.fori_loop(0, FMT_IT // 2, body, 0)
    pltpu.make_async_copy(ov0, wri_hbm.at[pl.ds(0, WIDE // 2)], so0).wait()
    pltpu.make_async_copy(ov1, wri_hbm.at[pl.ds(0, WIDE // 2)], so1).wait()

    @pl.when(wid == 0)
    def _tail():
        pltpu.sync_copy(wrt_hbm.at[:], av0.at[:, pl.ds(0, 128)])
        pltpu.sync_copy(wit_hbm.at[:], bv0.at[:, pl.ds(0, 128)])
        permute(av0, bv0, ov0, 8)
        pltpu.sync_copy(ov0.at[pl.ds(0, 64)],
                        wri_hbm.at[pl.ds((VOCAB - 128) // 2, 64)])


def _gather_body(ids_hbm, wri_hbm, out_hbm, idt, idx0, idx1, off0, off1,
                 gv0, gv1, ov0, ov1, sg0, sg1, so0, so1):
    wid = lax.axis_index("s") * NC + lax.axis_index("c")
    iota = lax.iota(jnp.int32, 16)
    idxs, offs = (idx0, idx1), (off0, off1)
    gvs, ovs = (gv0, gv1), (ov0, ov1)
    sgs, sos = (sg0, sg1), (so0, so1)

    def load_ids(i):
        pltpu.sync_copy(
            ids_hbm.at[pl.ds(8 * i, 8), pl.ds(wid * 128, 128)], idt)

    def prep_and_fire(u, p):
        @pl.when(u < NU)
        def _():
            hh = u & 7
            for g in range(8):
                t = idt[hh, pl.ds(16 * g, 16)]
                idxs[p][pl.ds(16 * g, 16)] = lax.shift_right_logical(t, 1)
                offs[p][pl.ds(16 * g, 16)] = (t & 1) * (2 * DIM)
            pltpu.async_copy(wri_hbm.at[idxs[p]], gvs[p], sgs[p])

    def interleave(p):
        def grp(g, c3):
            rows = g * 16 + iota
            off = offs[p][pl.ds(16 * g, 16)]
            for d0 in range(0, DIM, 8):
                vals = [plsc.load_gather(gvs[p], [rows, off + (c * DIM + d)])
                        for d in range(d0, d0 + 8) for c in range(2)]
                for j, (d, c) in enumerate(
                        (d, c) for d in range(d0, d0 + 8) for c in range(2)):
                    ovs[p][d, pl.ds(c * 128 + 16 * g, 16)] = vals[j]
            return c3
        lax.fori_loop(0, 8, grp, 0)

    def drain_out(p):
        pltpu.make_async_copy(ovs[p].at[pl.ds(0, 16)],
                              out_hbm.at[KB * iota], sos[p]).wait()
        pltpu.make_async_copy(ovs[p].at[pl.ds(16, 16)],
                              out_hbm.at[KB * iota], sos[p]).wait()

    load_ids(0)
    prep_and_fire(0, 0)

    def body(uu, carry):
        for p in (0, 1):
            u = 2 * uu + p
            nxt = u + 1

            @pl.when((nxt & 7) == 0)
            def _():
                @pl.when(nxt < NU)
                def __():
                    load_ids(lax.shift_right_logical(nxt, 3))

            prep_and_fire(nxt, (p + 1) % 2)
            pltpu.make_async_copy(wri_hbm.at[idxs[p]], gvs[p], sgs[p]).wait()

            @pl.when(u >= 2)
            def _():
                drain_out(p)

            interleave(p)
            base = u * (DIM * KB) + wid
            pltpu.async_copy(ovs[p].at[pl.ds(0, 16)],
                             out_hbm.at[base + KB * iota], sos[p])
            pltpu.async_copy(ovs[p].at[pl.ds(16, 16)],
                             out_hbm.at[base + 16 * KB + KB * iota], sos[p])
        return carry

    lax.fori_loop(0, NU // 2, body, 0)
    drain_out(0)
    drain_out(1)


@jax.jit
def _complex_embed(ids2, wr_t, wi_t, wr_tail, wi_tail):
    mesh = plsc.VectorSubcoreMesh(core_axis_name="c", subcore_axis_name="s")
    params = pltpu.CompilerParams(
        use_tc_tiling_on_sc=True, needs_layout_passes=False)

    wri = pl.kernel(
        _fmt_body,
        out_type=jax.ShapeDtypeStruct((QROWS, QDIM), jnp.float32),
        mesh=mesh,
        compiler_params=params,
        scratch_types=[
            pltpu.VMEM((DIM, WIDE), jnp.float32),
            pltpu.VMEM((DIM, WIDE), jnp.float32),
            pltpu.VMEM((DIM, WIDE), jnp.float32),
            pltpu.VMEM((DIM, WIDE), jnp.float32),
            pltpu.VMEM((WIDE // 2, QDIM), jnp.float32),
            pltpu.VMEM((WIDE // 2, QDIM), jnp.float32),
            pltpu.SemaphoreType.DMA,
            pltpu.SemaphoreType.DMA,
            pltpu.SemaphoreType.DMA,
            pltpu.SemaphoreType.DMA,
        ],
    )(wr_t, wi_t, wr_tail, wi_tail)

    out2d = pl.kernel(
        _gather_body,
        out_type=jax.ShapeDtypeStruct((OUT_ROWS, OUT_W), jnp.float32),
        mesh=mesh,
        compiler_params=params,
        scratch_types=[
            pltpu.VMEM((8, 128), jnp.int32),
            pltpu.VMEM((128,), jnp.int32),
            pltpu.VMEM((128,), jnp.int32),
            pltpu.VMEM((128,), jnp.int32),
            pltpu.VMEM((128,), jnp.int32),
            pltpu.VMEM((128, QDIM), jnp.float32),
            pltpu.VMEM((128, QDIM), jnp.float32),
            pltpu.VMEM((DIM, OUT_W), jnp.float32),
            pltpu.VMEM((DIM, OUT_W), jnp.float32),
            pltpu.SemaphoreType.DMA,
            pltpu.SemaphoreType.DMA,
            pltpu.SemaphoreType.DMA,
            pltpu.SemaphoreType.DMA,
        ],
    )(ids2, wri)
    return out2d


def kernel(token_ids, W_real, W_imag):
    wr_t = W_real.T
    wi_t = W_imag.T
    wr_tail = lax.slice(wr_t, (0, VOCAB - 128), (DIM, VOCAB))
    wi_tail = lax.slice(wi_t, (0, VOCAB - 128), (DIM, VOCAB))
    ids2 = token_ids.T.astype(jnp.int32)
    out2d = _complex_embed(ids2, wr_t, wi_t, wr_tail, wi_tail)
    out5 = out2d.reshape(HIST, DIM, KB, 2, 128)
    return out5.transpose(2, 4, 0, 1, 3).reshape(BATCH, HIST, DIM, 2)


# speedup vs baseline: 28.2625x; 3.1917x over previous
import functools

import jax
import jax.numpy as jnp
from jax import lax
from jax.experimental import pallas as pl
from jax.experimental.pallas import tpu as pltpu
from jax.experimental.pallas import tpu_sc as plsc

BATCH = 4096
HIST = 200
DIM = 32
VOCAB = 1000000
N = BATCH * HIST
NC = 2
NS = 16
NW = NC * NS
QROWS = VOCAB // 2
QDIM = 4 * DIM
WIDE = 256
NBLK = VOCAB // WIDE
FMT_IT = 124
HB = HIST // 8
KB = BATCH // 128
NU = HIST
OUT_ROWS = HIST * DIM * KB
OUT_W = 256


def _fmt_body(wr_hbm, wi_hbm, wrt_hbm, wit_hbm, wri_hbm,
              av0, av1, bv0, bv1, ov0, ov1, si0, si1, so0, so1):
    wid = lax.axis_index("s") * NC + lax.axis_index("c")
    iota = lax.iota(jnp.int32, 16)
    avs, bvs, ovs = (av0, av1), (bv0, bv1), (ov0, ov1)
    sis, sos = (si0, si1), (so0, so1)
    rowv, colv = [], []
    for v in range(WIDE // 16):
        l = iota + 16 * v
        rowv.append(lax.shift_right_logical(l, 1))
        colv.append((l & 1) * 64 + lax.shift_right_logical(l, 1))

    def issue_in(i, p):
        j = wid + NW * i

        @pl.when(j < NBLK)
        def _():
            pltpu.async_copy(wr_hbm.at[:, pl.ds(j * WIDE, WIDE)], avs[p],
                             sis[p])
            pltpu.async_copy(wi_hbm.at[:, pl.ds(j * WIDE, WIDE)], bvs[p],
                             sis[p])

    def permute(a, b, o, nv, qbase):
        def dloop(d, carry):
            ra = [a[d, pl.ds(16 * v, 16)] for v in range(nv)]
            rb = [b[d, pl.ds(16 * v, 16)] for v in range(nv)]
            for v in range(nv):
                plsc.store_scatter(
                    o, [rowv[v], (colv[v] + (d + qbase)) & 127], ra[v])
            for v in range(nv):
                plsc.store_scatter(
                    o, [rowv[v], (colv[v] + (d + DIM + qbase)) & 127], rb[v])
            return carry
        lax.fori_loop(0, DIM, dloop, 0)

    issue_in(0, 0)

    def body(ii, carry):
        for p in (0, 1):
            i = 2 * ii + p
            issue_in(i + 1, (p + 1) % 2)
            j = wid + NW * i

            @pl.when(j < NBLK)
            def _():
                pltpu.make_async_copy(
                    wr_hbm.at[:, pl.ds(0, WIDE)], avs[p], sis[p]).wait()
                pltpu.make_async_copy(
                    wi_hbm.at[:, pl.ds(0, WIDE)], bvs[p], sis[p]).wait()

                @pl.when(i >= 2)
                def __():
                    pltpu.make_async_copy(
                        ovs[p], wri_hbm.at[pl.ds(0, WIDE // 2)],
                        sos[p]).wait()

                permute(avs[p], bvs[p], ovs[p], WIDE // 16, 0)
                pltpu.async_copy(
                    ovs[p], wri_hbm.at[pl.ds(j * (WIDE // 2), WIDE // 2)],
                    sos[p])
        return carry

    lax.fori_loop(0, FMT_IT // 2, body, 0)
    pltpu.make_async_copy(ov0, wri_hbm.at[pl.ds(0, WIDE // 2)], so0).wait()
    pltpu.make_async_copy(ov1, wri_hbm.at[pl.ds(0, WIDE // 2)], so1).wait()

    @pl.when(wid == 0)
    def _tail():
        pltpu.sync_copy(wrt_hbm.at[:], av0.at[:, pl.ds(0, 128)])
        pltpu.sync_copy(wit_hbm.at[:], bv0.at[:, pl.ds(0, 128)])
        permute(av0, bv0, ov0, 8, ((VOCAB - 128) // 2) % 128)
        pltpu.sync_copy(ov0.at[pl.ds(0, 64)],
                        wri_hbm.at[pl.ds((VOCAB - 128) // 2, 64)])


def _gather_body(ids_hbm, wri_hbm, out_hbm, idt, idx0, idx1, off0, off1,
                 gv0, gv1, ov0, ov1, sg0, sg1, so0, so1):
    wid = lax.axis_index("s") * NC + lax.axis_index("c")
    iota = lax.iota(jnp.int32, 16)
    idxs, offs = (idx0, idx1), (off0, off1)
    gvs, ovs = (gv0, gv1), (ov0, ov1)
    sgs, sos = (sg0, sg1), (so0, so1)

    def load_ids(i):
        pltpu.sync_copy(
            ids_hbm.at[pl.ds(8 * i, 8), pl.ds(wid * 128, 128)], idt)

    def prep_and_fire(u, p):
        @pl.when(u < NU)
        def _():
            hh = u & 7
            for g in range(8):
                t = idt[hh, pl.ds(16 * g, 16)]
                sr = lax.shift_right_logical(t, 1)
                idxs[p][pl.ds(16 * g, 16)] = sr
                offs[p][pl.ds(16 * g, 16)] = (t & 1) * 64 + sr
            pltpu.async_copy(wri_hbm.at[idxs[p]], gvs[p], sgs[p])

    def interleave(p):
        def grp(g, c3):
            rows = g * 16 + iota
            off = offs[p][pl.ds(16 * g, 16)]
            for d0 in range(0, DIM, 8):
                vals = [plsc.load_gather(
                            gvs[p], [rows, (off + (c * DIM + d)) & 127])
                        for d in range(d0, d0 + 8) for c in range(2)]
                for j, (d, c) in enumerate(
                        (d, c) for d in range(d0, d0 + 8) for c in range(2)):
                    ovs[p][d, pl.ds(c * 128 + 16 * g, 16)] = vals[j]
            return c3
        lax.fori_loop(0, 8, grp, 0)

    def drain_out(p):
        pltpu.make_async_copy(ovs[p].at[pl.ds(0, 16)],
                              out_hbm.at[KB * iota], sos[p]).wait()
        pltpu.make_async_copy(ovs[p].at[pl.ds(16, 16)],
                              out_hbm.at[KB * iota], sos[p]).wait()

    load_ids(0)
    prep_and_fire(0, 0)

    def body(uu, carry):
        for p in (0, 1):
            u = 2 * uu + p
            nxt = u + 1

            @pl.when((nxt & 7) == 0)
            def _():
                @pl.when(nxt < NU)
                def __():
                    load_ids(lax.shift_right_logical(nxt, 3))

            prep_and_fire(nxt, (p + 1) % 2)
            pltpu.make_async_copy(wri_hbm.at[idxs[p]], gvs[p], sgs[p]).wait()

            @pl.when(u >= 2)
            def _():
                drain_out(p)

            interleave(p)
            base = u * (DIM * KB) + wid
            pltpu.async_copy(ovs[p].at[pl.ds(0, 16)],
                             out_hbm.at[base + KB * iota], sos[p])
            pltpu.async_copy(ovs[p].at[pl.ds(16, 16)],
                             out_hbm.at[base + 16 * KB + KB * iota], sos[p])
        return carry

    lax.fori_loop(0, NU // 2, body, 0)
    drain_out(0)
    drain_out(1)


@jax.jit
def _complex_embed(ids2, wr_t, wi_t, wr_tail, wi_tail):
    mesh = plsc.VectorSubcoreMesh(core_axis_name="c", subcore_axis_name="s")
    params = pltpu.CompilerParams(
        use_tc_tiling_on_sc=True, needs_layout_passes=False)

    wri = pl.kernel(
        _fmt_body,
        out_type=jax.ShapeDtypeStruct((QROWS, QDIM), jnp.float32),
        mesh=mesh,
        compiler_params=params,
        scratch_types=[
            pltpu.VMEM((DIM, WIDE), jnp.float32),
            pltpu.VMEM((DIM, WIDE), jnp.float32),
            pltpu.VMEM((DIM, WIDE), jnp.float32),
            pltpu.VMEM((DIM, WIDE), jnp.float32),
            pltpu.VMEM((WIDE // 2, QDIM), jnp.float32),
            pltpu.VMEM((WIDE // 2, QDIM), jnp.float32),
            pltpu.SemaphoreType.DMA,
            pltpu.SemaphoreType.DMA,
            pltpu.SemaphoreType.DMA,
            pltpu.SemaphoreType.DMA,
        ],
    )(wr_t, wi_t, wr_tail, wi_tail)

    out2d = pl.kernel(
        _gather_body,
        out_type=jax.ShapeDtypeStruct((OUT_ROWS, OUT_W), jnp.float32),
        mesh=mesh,
        compiler_params=params,
        scratch_types=[
            pltpu.VMEM((8, 128), jnp.int32),
            pltpu.VMEM((128,), jnp.int32),
            pltpu.VMEM((128,), jnp.int32),
            pltpu.VMEM((128,), jnp.int32),
            pltpu.VMEM((128,), jnp.int32),
            pltpu.VMEM((128, QDIM), jnp.float32),
            pltpu.VMEM((128, QDIM), jnp.float32),
            pltpu.VMEM((DIM, OUT_W), jnp.float32),
            pltpu.VMEM((DIM, OUT_W), jnp.float32),
            pltpu.SemaphoreType.DMA,
            pltpu.SemaphoreType.DMA,
            pltpu.SemaphoreType.DMA,
            pltpu.SemaphoreType.DMA,
        ],
    )(ids2, wri)
    return out2d


def kernel(token_ids, W_real, W_imag):
    wr_t = W_real.T
    wi_t = W_imag.T
    wr_tail = lax.slice(wr_t, (0, VOCAB - 128), (DIM, VOCAB))
    wi_tail = lax.slice(wi_t, (0, VOCAB - 128), (DIM, VOCAB))
    ids2 = token_ids.T.astype(jnp.int32)
    out2d = _complex_embed(ids2, wr_t, wi_t, wr_tail, wi_tail)
    out5 = out2d.reshape(HIST, DIM, KB, 2, 128)
    return out5.transpose(2, 4, 0, 1, 3).reshape(BATCH, HIST, DIM, 2)


# speedup vs baseline: 29.8275x; 1.0554x over previous
import functools

import jax
import jax.numpy as jnp
from jax import lax
from jax.experimental import pallas as pl
from jax.experimental.pallas import tpu as pltpu
from jax.experimental.pallas import tpu_sc as plsc

BATCH = 4096
HIST = 200
DIM = 32
VOCAB = 1000000
N = BATCH * HIST
NC = 2
NS = 16
NW = NC * NS
QROWS = VOCAB // 2
QDIM = 4 * DIM
WIDE = 256
NBLK = VOCAB // WIDE
FMT_IT = 124
HB = HIST // 8
KB = BATCH // 128
NU = HIST
OUT_ROWS = HIST * DIM * KB
OUT_W = 256


def _fmt_body(wr_hbm, wi_hbm, wrt_hbm, wit_hbm, wri_hbm,
              av0, av1, bv0, bv1, ov0, ov1, si0, si1, so0, so1):
    wid = lax.axis_index("s") * NC + lax.axis_index("c")
    iota = lax.iota(jnp.int32, 16)
    avs, bvs, ovs = (av0, av1), (bv0, bv1), (ov0, ov1)
    sis, sos = (si0, si1), (so0, so1)
    rowv, colv = [], []
    for v in range(WIDE // 16):
        l = iota + 16 * v
        rowv.append(lax.shift_right_logical(l, 1))
        colv.append((l & 1) * 64 + lax.shift_right_logical(l, 1))

    def issue_in(i, p):
        j = wid + NW * i

        @pl.when(j < NBLK)
        def _():
            pltpu.async_copy(wr_hbm.at[:, pl.ds(j * WIDE, WIDE)], avs[p],
                             sis[p])
            pltpu.async_copy(wi_hbm.at[:, pl.ds(j * WIDE, WIDE)], bvs[p],
                             sis[p])

    def permute(a, b, o, nv, qbase):
        def dloop(d, carry):
            ra = [a[d, pl.ds(16 * v, 16)] for v in range(nv)]
            rb = [b[d, pl.ds(16 * v, 16)] for v in range(nv)]
            for v in range(nv):
                plsc.store_scatter(
                    o, [rowv[v], (colv[v] + (d + qbase)) & 127], ra[v])
            for v in range(nv):
                plsc.store_scatter(
                    o, [rowv[v], (colv[v] + (d + DIM + qbase)) & 127], rb[v])
            return carry
        lax.fori_loop(0, DIM, dloop, 0)

    issue_in(0, 0)

    def body(ii, carry):
        for p in (0, 1):
            i = 2 * ii + p
            issue_in(i + 1, (p + 1) % 2)
            j = wid + NW * i

            @pl.when(j < NBLK)
            def _():
                pltpu.make_async_copy(
                    wr_hbm.at[:, pl.ds(0, WIDE)], avs[p], sis[p]).wait()
                pltpu.make_async_copy(
                    wi_hbm.at[:, pl.ds(0, WIDE)], bvs[p], sis[p]).wait()

                @pl.when(i >= 2)
                def __():
                    pltpu.make_async_copy(
                        ovs[p], wri_hbm.at[pl.ds(0, WIDE // 2)],
                        sos[p]).wait()

                permute(avs[p], bvs[p], ovs[p], WIDE // 16, 0)
                pltpu.async_copy(
                    ovs[p], wri_hbm.at[pl.ds(j * (WIDE // 2), WIDE // 2)],
                    sos[p])
        return carry

    lax.fori_loop(0, FMT_IT // 2, body, 0)
    pltpu.make_async_copy(ov0, wri_hbm.at[pl.ds(0, WIDE // 2)], so0).wait()
    pltpu.make_async_copy(ov1, wri_hbm.at[pl.ds(0, WIDE // 2)], so1).wait()

    @pl.when(wid == 0)
    def _tail():
        pltpu.sync_copy(wrt_hbm.at[:], av0.at[:, pl.ds(0, 128)])
        pltpu.sync_copy(wit_hbm.at[:], bv0.at[:, pl.ds(0, 128)])
        permute(av0, bv0, ov0, 8, ((VOCAB - 128) // 2) % 128)
        pltpu.sync_copy(ov0.at[pl.ds(0, 64)],
                        wri_hbm.at[pl.ds((VOCAB - 128) // 2, 64)])


def _gather_body(ids_hbm, wri_hbm, out_hbm, idt, idx0, idx1, idx2,
                 off0, off1, off2, gv0, gv1, gv2, ov0, ov1, ov2,
                 sg0, sg1, sg2, so0, so1, so2):
    wid = lax.axis_index("s") * NC + lax.axis_index("c")
    iota = lax.iota(jnp.int32, 16)
    idxs, offs = (idx0, idx1, idx2), (off0, off1, off2)
    gvs, ovs = (gv0, gv1, gv2), (ov0, ov1, ov2)
    sgs, sos = (sg0, sg1, sg2), (so0, so1, so2)

    def load_ids(i):
        pltpu.sync_copy(
            ids_hbm.at[pl.ds(8 * i, 8), pl.ds(wid * 128, 128)], idt)

    def prep_and_fire(u, p):
        @pl.when(u < NU)
        def _():
            hh = u & 7
            for g in range(8):
                t = idt[hh, pl.ds(16 * g, 16)]
                sr = lax.shift_right_logical(t, 1)
                idxs[p][pl.ds(16 * g, 16)] = sr
                offs[p][pl.ds(16 * g, 16)] = (t & 1) * 64 + sr
            pltpu.async_copy(wri_hbm.at[idxs[p]], gvs[p], sgs[p])

    def interleave(p):
        def grp(g, c3):
            rows = g * 16 + iota
            off = offs[p][pl.ds(16 * g, 16)]
            for d0 in range(0, DIM, 8):
                vals = [plsc.load_gather(
                            gvs[p], [rows, (off + (c * DIM + d)) & 127])
                        for d in range(d0, d0 + 8) for c in range(2)]
                for j, (d, c) in enumerate(
                        (d, c) for d in range(d0, d0 + 8) for c in range(2)):
                    ovs[p][d, pl.ds(c * 128 + 16 * g, 16)] = vals[j]
            return c3
        lax.fori_loop(0, 8, grp, 0)

    def drain_out(p):
        pltpu.make_async_copy(ovs[p].at[pl.ds(0, 16)],
                              out_hbm.at[KB * iota], sos[p]).wait()
        pltpu.make_async_copy(ovs[p].at[pl.ds(16, 16)],
                              out_hbm.at[KB * iota], sos[p]).wait()

    load_ids(0)
    prep_and_fire(0, 0)
    prep_and_fire(1, 1)

    def body(uu, carry):
        for p in (0, 1, 2):
            u = 3 * uu + p
            nxt = u + 2

            @pl.when((nxt & 7) == 0)
            def _():
                @pl.when(nxt < NU)
                def __():
                    load_ids(lax.shift_right_logical(nxt, 3))

            prep_and_fire(nxt, (p + 2) % 3)

            @pl.when(u < NU)
            def _():
                pltpu.make_async_copy(
                    wri_hbm.at[idxs[p]], gvs[p], sgs[p]).wait()

                @pl.when(u >= 3)
                def __():
                    drain_out(p)

                interleave(p)
                base = u * (DIM * KB) + wid
                pltpu.async_copy(ovs[p].at[pl.ds(0, 16)],
                                 out_hbm.at[base + KB * iota], sos[p])
                pltpu.async_copy(ovs[p].at[pl.ds(16, 16)],
                                 out_hbm.at[base + 16 * KB + KB * iota],
                                 sos[p])
        return carry

    lax.fori_loop(0, (NU + 2) // 3, body, 0)
    drain_out(0)
    drain_out(1)
    drain_out(2)


@jax.jit
def _complex_embed(ids2, wr_t, wi_t, wr_tail, wi_tail):
    mesh = plsc.VectorSubcoreMesh(core_axis_name="c", subcore_axis_name="s")
    params = pltpu.CompilerParams(
        use_tc_tiling_on_sc=True, needs_layout_passes=False)

    wri = pl.kernel(
        _fmt_body,
        out_type=jax.ShapeDtypeStruct((QROWS, QDIM), jnp.float32),
        mesh=mesh,
        compiler_params=params,
        scratch_types=[
            pltpu.VMEM((DIM, WIDE), jnp.float32),
            pltpu.VMEM((DIM, WIDE), jnp.float32),
            pltpu.VMEM((DIM, WIDE), jnp.float32),
            pltpu.VMEM((DIM, WIDE), jnp.float32),
            pltpu.VMEM((WIDE // 2, QDIM), jnp.float32),
            pltpu.VMEM((WIDE // 2, QDIM), jnp.float32),
            pltpu.SemaphoreType.DMA,
            pltpu.SemaphoreType.DMA,
            pltpu.SemaphoreType.DMA,
            pltpu.SemaphoreType.DMA,
        ],
    )(wr_t, wi_t, wr_tail, wi_tail)

    out2d = pl.kernel(
        _gather_body,
        out_type=jax.ShapeDtypeStruct((OUT_ROWS, OUT_W), jnp.float32),
        mesh=mesh,
        compiler_params=params,
        scratch_types=[
            pltpu.VMEM((8, 128), jnp.int32),
            pltpu.VMEM((128,), jnp.int32),
            pltpu.VMEM((128,), jnp.int32),
            pltpu.VMEM((128,), jnp.int32),
            pltpu.VMEM((128,), jnp.int32),
            pltpu.VMEM((128,), jnp.int32),
            pltpu.VMEM((128,), jnp.int32),
            pltpu.VMEM((128, QDIM), jnp.float32),
            pltpu.VMEM((128, QDIM), jnp.float32),
            pltpu.VMEM((128, QDIM), jnp.float32),
            pltpu.VMEM((DIM, OUT_W), jnp.float32),
            pltpu.VMEM((DIM, OUT_W), jnp.float32),
            pltpu.VMEM((DIM, OUT_W), jnp.float32),
            pltpu.SemaphoreType.DMA,
            pltpu.SemaphoreType.DMA,
            pltpu.SemaphoreType.DMA,
            pltpu.SemaphoreType.DMA,
            pltpu.SemaphoreType.DMA,
            pltpu.SemaphoreType.DMA,
        ],
    )(ids2, wri)
    return out2d


def kernel(token_ids, W_real, W_imag):
    wr_t = W_real.T
    wi_t = W_imag.T
    wr_tail = lax.slice(wr_t, (0, VOCAB - 128), (DIM, VOCAB))
    wi_tail = lax.slice(wi_t, (0, VOCAB - 128), (DIM, VOCAB))
    ids2 = token_ids.T.astype(jnp.int32)
    out2d = _complex_embed(ids2, wr_t, wi_t, wr_tail, wi_tail)
    out5 = out2d.reshape(HIST, DIM, KB, 2, 128)
    return out5.transpose(2, 4, 0, 1, 3).reshape(BATCH, HIST, DIM, 2)


# speedup vs baseline: 30.6529x; 1.0277x over previous
import functools

import jax
import jax.numpy as jnp
from jax import lax
from jax.experimental import pallas as pl
from jax.experimental.pallas import tpu as pltpu
from jax.experimental.pallas import tpu_sc as plsc

BATCH = 4096
HIST = 200
DIM = 32
VOCAB = 1000000
N = BATCH * HIST
NC = 2
NS = 16
NW = NC * NS
QROWS = VOCAB // 2
QDIM = 4 * DIM
WIDE = 256
NBLK = VOCAB // WIDE
FMT_IT = 124
HB = HIST // 8
KB = BATCH // 128
NU = HIST
OUT_ROWS = HIST * DIM * KB
OUT_W = 256


def _fmt_body(wr_hbm, wi_hbm, wrt_hbm, wit_hbm, wri_hbm,
              av0, av1, av2, bv0, bv1, bv2, ov0, ov1, ov2,
              si0, si1, si2, so0, so1, so2):
    wid = lax.axis_index("s") * NC + lax.axis_index("c")
    iota = lax.iota(jnp.int32, 16)
    avs, bvs, ovs = (av0, av1, av2), (bv0, bv1, bv2), (ov0, ov1, ov2)
    sis, sos = (si0, si1, si2), (so0, so1, so2)
    rowv, colv = [], []
    for v in range(WIDE // 16):
        l = iota + 16 * v
        rowv.append(lax.shift_right_logical(l, 1))
        colv.append((l & 1) * 64 + lax.shift_right_logical(l, 1))

    def issue_in(i, p):
        j = wid + NW * i

        @pl.when(j < NBLK)
        def _():
            pltpu.async_copy(wr_hbm.at[:, pl.ds(j * WIDE, WIDE)], avs[p],
                             sis[p])
            pltpu.async_copy(wi_hbm.at[:, pl.ds(j * WIDE, WIDE)], bvs[p],
                             sis[p])

    def permute(a, b, o, nv, qbase):
        def dloop(d, carry):
            ra = [a[d, pl.ds(16 * v, 16)] for v in range(nv)]
            rb = [b[d, pl.ds(16 * v, 16)] for v in range(nv)]
            for v in range(nv):
                plsc.store_scatter(
                    o, [rowv[v], (colv[v] + (d + qbase)) & 127], ra[v])
            for v in range(nv):
                plsc.store_scatter(
                    o, [rowv[v], (colv[v] + (d + DIM + qbase)) & 127], rb[v])
            return carry
        lax.fori_loop(0, DIM, dloop, 0)

    issue_in(0, 0)
    issue_in(1, 1)

    def body(ii, carry):
        for p in (0, 1, 2):
            i = 3 * ii + p
            issue_in(i + 2, (p + 2) % 3)
            j = wid + NW * i

            @pl.when(j < NBLK)
            def _():
                pltpu.make_async_copy(
                    wr_hbm.at[:, pl.ds(0, WIDE)], avs[p], sis[p]).wait()
                pltpu.make_async_copy(
                    wi_hbm.at[:, pl.ds(0, WIDE)], bvs[p], sis[p]).wait()

                @pl.when(i >= 3)
                def __():
                    pltpu.make_async_copy(
                        ovs[p], wri_hbm.at[pl.ds(0, WIDE // 2)],
                        sos[p]).wait()

                permute(avs[p], bvs[p], ovs[p], WIDE // 16, 0)
                pltpu.async_copy(
                    ovs[p], wri_hbm.at[pl.ds(j * (WIDE // 2), WIDE // 2)],
                    sos[p])
        return carry

    lax.fori_loop(0, (FMT_IT + 2) // 3, body, 0)
    pltpu.make_async_copy(ov0, wri_hbm.at[pl.ds(0, WIDE // 2)], so0).wait()
    pltpu.make_async_copy(ov1, wri_hbm.at[pl.ds(0, WIDE // 2)], so1).wait()
    pltpu.make_async_copy(ov2, wri_hbm.at[pl.ds(0, WIDE // 2)], so2).wait()

    @pl.when(wid == 0)
    def _tail():
        pltpu.sync_copy(wrt_hbm.at[:], av0.at[:, pl.ds(0, 128)])
        pltpu.sync_copy(wit_hbm.at[:], bv0.at[:, pl.ds(0, 128)])
        permute(av0, bv0, ov0, 8, ((VOCAB - 128) // 2) % 128)
        pltpu.sync_copy(ov0.at[pl.ds(0, 64)],
                        wri_hbm.at[pl.ds((VOCAB - 128) // 2, 64)])


def _gather_body(ids_hbm, wri_hbm, out_hbm, idt, idx0, idx1, idx2,
                 off0, off1, off2, gv0, gv1, gv2, ov0, ov1, ov2,
                 sg0, sg1, sg2, so0, so1, so2):
    wid = lax.axis_index("s") * NC + lax.axis_index("c")
    iota = lax.iota(jnp.int32, 16)
    idxs, offs = (idx0, idx1, idx2), (off0, off1, off2)
    gvs, ovs = (gv0, gv1, gv2), (ov0, ov1, ov2)
    sgs, sos = (sg0, sg1, sg2), (so0, so1, so2)

    def load_ids(i):
        pltpu.sync_copy(
            ids_hbm.at[pl.ds(8 * i, 8), pl.ds(wid * 128, 128)], idt)

    def prep_and_fire(u, p):
        @pl.when(u < NU)
        def _():
            hh = u & 7
            for g in range(8):
                t = idt[hh, pl.ds(16 * g, 16)]
                sr = lax.shift_right_logical(t, 1)
                idxs[p][pl.ds(16 * g, 16)] = sr
                offs[p][pl.ds(16 * g, 16)] = (t & 1) * 64 + sr
            pltpu.async_copy(wri_hbm.at[idxs[p]], gvs[p], sgs[p])

    def interleave(p):
        def grp(g, c3):
            rows = g * 16 + iota
            off = offs[p][pl.ds(16 * g, 16)]
            for d0 in range(0, DIM, 8):
                vals = [plsc.load_gather(
                            gvs[p], [rows, (off + (c * DIM + d)) & 127])
                        for d in range(d0, d0 + 8) for c in range(2)]
                for j, (d, c) in enumerate(
                        (d, c) for d in range(d0, d0 + 8) for c in range(2)):
                    ovs[p][d, pl.ds(c * 128 + 16 * g, 16)] = vals[j]
            return c3
        lax.fori_loop(0, 8, grp, 0)

    def drain_out(p):
        pltpu.make_async_copy(ovs[p].at[pl.ds(0, 16)],
                              out_hbm.at[KB * iota], sos[p]).wait()
        pltpu.make_async_copy(ovs[p].at[pl.ds(16, 16)],
                              out_hbm.at[KB * iota], sos[p]).wait()

    load_ids(0)
    prep_and_fire(0, 0)
    prep_and_fire(1, 1)

    def body(uu, carry):
        for p in (0, 1, 2):
            u = 3 * uu + p
            nxt = u + 2

            @pl.when((nxt & 7) == 0)
            def _():
                @pl.when(nxt < NU)
                def __():
                    load_ids(lax.shift_right_logical(nxt, 3))

            prep_and_fire(nxt, (p + 2) % 3)

            @pl.when(u < NU)
            def _():
                pltpu.make_async_copy(
                    wri_hbm.at[idxs[p]], gvs[p], sgs[p]).wait()

                @pl.when(u >= 3)
                def __():
                    drain_out(p)

                interleave(p)
                base = u * (DIM * KB) + wid
                pltpu.async_copy(ovs[p].at[pl.ds(0, 16)],
                                 out_hbm.at[base + KB * iota], sos[p])
                pltpu.async_copy(ovs[p].at[pl.ds(16, 16)],
                                 out_hbm.at[base + 16 * KB + KB * iota],
                                 sos[p])
        return carry

    lax.fori_loop(0, (NU + 2) // 3, body, 0)
    drain_out(0)
    drain_out(1)
    drain_out(2)


@jax.jit
def _complex_embed(ids2, wr_t, wi_t, wr_tail, wi_tail):
    mesh = plsc.VectorSubcoreMesh(core_axis_name="c", subcore_axis_name="s")
    params = pltpu.CompilerParams(
        use_tc_tiling_on_sc=True, needs_layout_passes=False)

    wri = pl.kernel(
        _fmt_body,
        out_type=jax.ShapeDtypeStruct((QROWS, QDIM), jnp.float32),
        mesh=mesh,
        compiler_params=params,
        scratch_types=[
            pltpu.VMEM((DIM, WIDE), jnp.float32),
            pltpu.VMEM((DIM, WIDE), jnp.float32),
            pltpu.VMEM((DIM, WIDE), jnp.float32),
            pltpu.VMEM((DIM, WIDE), jnp.float32),
            pltpu.VMEM((DIM, WIDE), jnp.float32),
            pltpu.VMEM((DIM, WIDE), jnp.float32),
            pltpu.VMEM((WIDE // 2, QDIM), jnp.float32),
            pltpu.VMEM((WIDE // 2, QDIM), jnp.float32),
            pltpu.VMEM((WIDE // 2, QDIM), jnp.float32),
            pltpu.SemaphoreType.DMA,
            pltpu.SemaphoreType.DMA,
            pltpu.SemaphoreType.DMA,
            pltpu.SemaphoreType.DMA,
            pltpu.SemaphoreType.DMA,
            pltpu.SemaphoreType.DMA,
        ],
    )(wr_t, wi_t, wr_tail, wi_tail)

    out2d = pl.kernel(
        _gather_body,
        out_type=jax.ShapeDtypeStruct((OUT_ROWS, OUT_W), jnp.float32),
        mesh=mesh,
        compiler_params=params,
        scratch_types=[
            pltpu.VMEM((8, 128), jnp.int32),
            pltpu.VMEM((128,), jnp.int32),
            pltpu.VMEM((128,), jnp.int32),
            pltpu.VMEM((128,), jnp.int32),
            pltpu.VMEM((128,), jnp.int32),
            pltpu.VMEM((128,), jnp.int32),
            pltpu.VMEM((128,), jnp.int32),
            pltpu.VMEM((128, QDIM), jnp.float32),
            pltpu.VMEM((128, QDIM), jnp.float32),
            pltpu.VMEM((128, QDIM), jnp.float32),
            pltpu.VMEM((DIM, OUT_W), jnp.float32),
            pltpu.VMEM((DIM, OUT_W), jnp.float32),
            pltpu.VMEM((DIM, OUT_W), jnp.float32),
            pltpu.SemaphoreType.DMA,
            pltpu.SemaphoreType.DMA,
            pltpu.SemaphoreType.DMA,
            pltpu.SemaphoreType.DMA,
            pltpu.SemaphoreType.DMA,
            pltpu.SemaphoreType.DMA,
        ],
    )(ids2, wri)
    return out2d


def kernel(token_ids, W_real, W_imag):
    wr_t = W_real.T
    wi_t = W_imag.T
    wr_tail = lax.slice(wr_t, (0, VOCAB - 128), (DIM, VOCAB))
    wi_tail = lax.slice(wi_t, (0, VOCAB - 128), (DIM, VOCAB))
    ids2 = token_ids.T.astype(jnp.int32)
    out2d = _complex_embed(ids2, wr_t, wi_t, wr_tail, wi_tail)
    out5 = out2d.reshape(HIST, DIM, KB, 2, 128)
    return out5.transpose(2, 4, 0, 1, 3).reshape(BATCH, HIST, DIM, 2)


# speedup vs baseline: 30.7198x; 1.0022x over previous
import jax
import jax.numpy as jnp
from jax import lax
from jax.experimental import pallas as pl
from jax.experimental.pallas import tpu as pltpu
from jax.experimental.pallas import tpu_sc as plsc

BATCH = 4096
HIST = 200
DIM = 32
VOCAB = 1000000
N = BATCH * HIST
NC = 2
NS = 16
NW = NC * NS
QROWS = VOCAB // 2
QDIM = 4 * DIM
WIDE = 256
NBLK = VOCAB // WIDE
FMT_IT = 124
HB = HIST // 8
KB = BATCH // 128
NU = HIST
OUT_ROWS = HIST * DIM * KB
OUT_W = 256


def _fmt_body(wr_hbm, wi_hbm, wrt_hbm, wit_hbm, wri_hbm,
              av0, av1, av2, bv0, bv1, bv2, ov0, ov1, ov2,
              si0, si1, si2, so0, so1, so2):
    wid = lax.axis_index("s") * NC + lax.axis_index("c")
    iota = lax.iota(jnp.int32, 16)
    avs, bvs, ovs = (av0, av1, av2), (bv0, bv1, bv2), (ov0, ov1, ov2)
    sis, sos = (si0, si1, si2), (so0, so1, so2)
    rowv, colv = [], []
    for v in range(WIDE // 16):
        l = iota + 16 * v
        rowv.append(lax.shift_right_logical(l, 1))
        colv.append((l & 1) * 64 + lax.shift_right_logical(l, 1))

    def issue_in(i, p):
        j = wid + NW * i

        @pl.when(j < NBLK)
        def _():
            pltpu.async_copy(wr_hbm.at[:, pl.ds(j * WIDE, WIDE)], avs[p],
                             sis[p])
            pltpu.async_copy(wi_hbm.at[:, pl.ds(j * WIDE, WIDE)], bvs[p],
                             sis[p])

    def permute(a, b, o, nv, qbase):
        def dloop(d, carry):
            ra = [a[d, pl.ds(16 * v, 16)] for v in range(nv)]
            rb = [b[d, pl.ds(16 * v, 16)] for v in range(nv)]
            for v in range(nv):
                plsc.store_scatter(
                    o, [rowv[v], (colv[v] + (d + qbase)) & 127], ra[v])
            for v in range(nv):
                plsc.store_scatter(
                    o, [rowv[v], (colv[v] + (d + DIM + qbase)) & 127], rb[v])
            return carry
        lax.fori_loop(0, DIM, dloop, 0)

    issue_in(0, 0)
    issue_in(1, 1)

    def body(ii, carry):
        for p in (0, 1, 2):
            i = 3 * ii + p
            issue_in(i + 2, (p + 2) % 3)
            j = wid + NW * i

            @pl.when(j < NBLK)
            def _():
                pltpu.make_async_copy(
                    wr_hbm.at[:, pl.ds(0, WIDE)], avs[p], sis[p]).wait()
                pltpu.make_async_copy(
                    wi_hbm.at[:, pl.ds(0, WIDE)], bvs[p], sis[p]).wait()

                @pl.when(i >= 3)
                def __():
                    pltpu.make_async_copy(
                        ovs[p], wri_hbm.at[pl.ds(0, WIDE // 2)],
                        sos[p]).wait()

                permute(avs[p], bvs[p], ovs[p], WIDE // 16, 0)
                pltpu.async_copy(
                    ovs[p], wri_hbm.at[pl.ds(j * (WIDE // 2), WIDE // 2)],
                    sos[p])
        return carry

    lax.fori_loop(0, (FMT_IT + 2) // 3, body, 0)
    pltpu.make_async_copy(ov0, wri_hbm.at[pl.ds(0, WIDE // 2)], so0).wait()
    pltpu.make_async_copy(ov1, wri_hbm.at[pl.ds(0, WIDE // 2)], so1).wait()
    pltpu.make_async_copy(ov2, wri_hbm.at[pl.ds(0, WIDE // 2)], so2).wait()

    @pl.when(wid == 0)
    def _tail():
        pltpu.sync_copy(wrt_hbm.at[:], av0.at[:, pl.ds(0, 128)])
        pltpu.sync_copy(wit_hbm.at[:], bv0.at[:, pl.ds(0, 128)])
        permute(av0, bv0, ov0, 8, ((VOCAB - 128) // 2) % 128)
        pltpu.sync_copy(ov0.at[pl.ds(0, 64)],
                        wri_hbm.at[pl.ds((VOCAB - 128) // 2, 64)])


def _gather_body(ids_hbm, wri_hbm, out_hbm, idt, idx0, idx1, idx2,
                 off0, off1, off2, gv0, gv1, gv2, ov0, ov1, ov2,
                 sg0, sg1, sg2, so0, so1, so2):
    wid = lax.axis_index("s") * NC + lax.axis_index("c")
    iota = lax.iota(jnp.int32, 16)
    idxs, offs = (idx0, idx1, idx2), (off0, off1, off2)
    gvs, ovs = (gv0, gv1, gv2), (ov0, ov1, ov2)
    sgs, sos = (sg0, sg1, sg2), (so0, so1, so2)

    def load_ids(i):
        pltpu.sync_copy(
            ids_hbm.at[pl.ds(8 * i, 8), pl.ds(wid * 128, 128)], idt)

    def prep_and_fire(u, p):
        @pl.when(u < NU)
        def _():
            hh = u & 7
            for g in range(8):
                t = idt[hh, pl.ds(16 * g, 16)]
                sr = lax.shift_right_logical(t, 1)
                idxs[p][pl.ds(16 * g, 16)] = sr
                offs[p][pl.ds(16 * g, 16)] = (t & 1) * 64 + sr
            pltpu.async_copy(wri_hbm.at[idxs[p]], gvs[p], sgs[p])

    def interleave(p):
        def grp(g, c3):
            rows = g * 16 + iota
            off = offs[p][pl.ds(16 * g, 16)]
            for d0 in range(0, DIM, 8):
                vals = [plsc.load_gather(
                            gvs[p], [rows, (off + (c * DIM + d)) & 127])
                        for d in range(d0, d0 + 8) for c in range(2)]
                for j, (d, c) in enumerate(
                        (d, c) for d in range(d0, d0 + 8) for c in range(2)):
                    ovs[p][d, pl.ds(c * 128 + 16 * g, 16)] = vals[j]
            return c3
        lax.fori_loop(0, 8, grp, 0)

    def drain_out(p):
        pltpu.make_async_copy(ovs[p].at[pl.ds(0, 16)],
                              out_hbm.at[KB * iota], sos[p]).wait()
        pltpu.make_async_copy(ovs[p].at[pl.ds(16, 16)],
                              out_hbm.at[KB * iota], sos[p]).wait()

    load_ids(0)
    prep_and_fire(0, 0)
    prep_and_fire(1, 1)

    def body(uu, carry):
        for p in (0, 1, 2):
            u = 3 * uu + p
            nxt = u + 2

            @pl.when((nxt & 7) == 0)
            def _():
                @pl.when(nxt < NU)
                def __():
                    load_ids(lax.shift_right_logical(nxt, 3))

            prep_and_fire(nxt, (p + 2) % 3)

            @pl.when(u < NU)
            def _():
                pltpu.make_async_copy(
                    wri_hbm.at[idxs[p]], gvs[p], sgs[p]).wait()

                @pl.when(u >= 3)
                def __():
                    drain_out(p)

                interleave(p)
                base = u * (DIM * KB) + wid
                pltpu.async_copy(ovs[p].at[pl.ds(0, 16)],
                                 out_hbm.at[base + KB * iota], sos[p])
                pltpu.async_copy(ovs[p].at[pl.ds(16, 16)],
                                 out_hbm.at[base + 16 * KB + KB * iota],
                                 sos[p])
        return carry

    lax.fori_loop(0, (NU + 2) // 3, body, 0)
    drain_out(0)
    drain_out(1)
    drain_out(2)


@jax.jit
def _complex_embed(ids2, wr_t, wi_t, wr_tail, wi_tail):
    mesh = plsc.VectorSubcoreMesh(core_axis_name="c", subcore_axis_name="s")
    params = pltpu.CompilerParams(
        use_tc_tiling_on_sc=True, needs_layout_passes=False)

    wri = pl.kernel(
        _fmt_body,
        out_type=jax.ShapeDtypeStruct((QROWS, QDIM), jnp.float32),
        mesh=mesh,
        compiler_params=params,
        scratch_types=[
            pltpu.VMEM((DIM, WIDE), jnp.float32),
            pltpu.VMEM((DIM, WIDE), jnp.float32),
            pltpu.VMEM((DIM, WIDE), jnp.float32),
            pltpu.VMEM((DIM, WIDE), jnp.float32),
            pltpu.VMEM((DIM, WIDE), jnp.float32),
            pltpu.VMEM((DIM, WIDE), jnp.float32),
            pltpu.VMEM((WIDE // 2, QDIM), jnp.float32),
            pltpu.VMEM((WIDE // 2, QDIM), jnp.float32),
            pltpu.VMEM((WIDE // 2, QDIM), jnp.float32),
            pltpu.SemaphoreType.DMA,
            pltpu.SemaphoreType.DMA,
            pltpu.SemaphoreType.DMA,
            pltpu.SemaphoreType.DMA,
            pltpu.SemaphoreType.DMA,
            pltpu.SemaphoreType.DMA,
        ],
    )(wr_t, wi_t, wr_tail, wi_tail)

    out2d = pl.kernel(
        _gather_body,
        out_type=jax.ShapeDtypeStruct((OUT_ROWS, OUT_W), jnp.float32),
        mesh=mesh,
        compiler_params=params,
        scratch_types=[
            pltpu.VMEM((8, 128), jnp.int32),
            pltpu.VMEM((128,), jnp.int32),
            pltpu.VMEM((128,), jnp.int32),
            pltpu.VMEM((128,), jnp.int32),
            pltpu.VMEM((128,), jnp.int32),
            pltpu.VMEM((128,), jnp.int32),
            pltpu.VMEM((128,), jnp.int32),
            pltpu.VMEM((128, QDIM), jnp.float32),
            pltpu.VMEM((128, QDIM), jnp.float32),
            pltpu.VMEM((128, QDIM), jnp.float32),
            pltpu.VMEM((DIM, OUT_W), jnp.float32),
            pltpu.VMEM((DIM, OUT_W), jnp.float32),
            pltpu.VMEM((DIM, OUT_W), jnp.float32),
            pltpu.SemaphoreType.DMA,
            pltpu.SemaphoreType.DMA,
            pltpu.SemaphoreType.DMA,
            pltpu.SemaphoreType.DMA,
            pltpu.SemaphoreType.DMA,
            pltpu.SemaphoreType.DMA,
        ],
    )(ids2, wri)
    return out2d


def kernel(token_ids, W_real, W_imag):
    wr_t = W_real.T
    wi_t = W_imag.T
    wr_tail = lax.slice(wr_t, (0, VOCAB - 128), (DIM, VOCAB))
    wi_tail = lax.slice(wi_t, (0, VOCAB - 128), (DIM, VOCAB))
    ids2 = token_ids.T.astype(jnp.int32)
    out2d = _complex_embed(ids2, wr_t, wi_t, wr_tail, wi_tail)
    out5 = out2d.reshape(HIST, DIM, KB, 2, 128)
    return out5.transpose(2, 4, 0, 1, 3).reshape(BATCH, HIST, DIM, 2)
